# Initial kernel scaffold; baseline (speedup 1.0000x reference)
import jax
import jax.numpy as jnp
from jax.experimental import pallas as pl


def kernel(x, W, U):
    raise NotImplementedError("write your pallas kernel here")


# speedup vs baseline: 9.6995x; 9.6995x over previous
import jax
import jax.numpy as jnp
from jax import lax
from jax.experimental import pallas as pl

TOPK_NUM = 64
TEMP = 0.1
EPS = 1e-20


def _fused_body(x_ref, w_ref, u_ref, out_ref):
    x = x_ref[0]
    W = w_ref[...]
    U = u_ref[0]

    mapping = jax.lax.dot_general(
        W, x, (((1,), (0,)), ((), ())), preferred_element_type=jnp.float32
    )

    g = -jnp.log(-jnp.log(U + EPS) + EPS)
    z = mapping + g

    rmax = jnp.max(z, axis=1, keepdims=True)
    col = lax.broadcasted_iota(jnp.int32, z.shape, 1)
    big = jnp.int32(1 << 30)
    idx = jnp.min(jnp.where(z == rmax, col, big), axis=1, keepdims=True)
    onehot = (col == idx).astype(jnp.float32)

    xc = jax.lax.dot_general(
        onehot, x, (((1,), (1,)), ((), ())), preferred_element_type=jnp.float32
    )

    mmax = jnp.max(mapping, axis=0, keepdims=True)
    e = jnp.exp(mapping - mmax)
    mp = e / jnp.sum(e, axis=0, keepdims=True)

    out_ref[0] = jax.lax.dot_general(
        xc, mp, (((0,), (0,)), ((), ())), preferred_element_type=jnp.float32
    )


def kernel(x, W, U):
    n, c, h, w = x.shape
    k = W.shape[0]
    hw = h * w
    x2 = x.reshape(n, c, hw)
    U2 = U.reshape(n, k, hw)

    out = pl.pallas_call(
        _fused_body,
        grid=(n,),
        in_specs=[
            pl.BlockSpec((1, c, hw), lambda i: (i, 0, 0)),
            pl.BlockSpec((k, c), lambda i: (0, 0)),
            pl.BlockSpec((1, k, hw), lambda i: (i, 0, 0)),
        ],
        out_specs=pl.BlockSpec((1, c, hw), lambda i: (i, 0, 0)),
        out_shape=jax.ShapeDtypeStruct((n, c, hw), jnp.float32),
    )(x2, W, U2)
    return out.reshape(n, c, h, w)
